# baseline (speedup 1.0000x reference)
import jax
import jax.numpy as jnp
from jax.experimental import pallas as pl
from jax.experimental.pallas import tpu as pltpu

_BM1 = 128
_BM2 = 256
_CA_DTYPE = jnp.float32


def _p1_body(meta_ref, x_ref, w1_ref, af_ref, as_ref, z1_ref, ca_ref, t1_scr):
    @pl.when(pl.program_id(0) == 0)
    def _():
        t1_scr[...] = jnp.tanh(
            jnp.dot(x_ref[...], w1_ref[...], preferred_element_type=jnp.float32))

    m = meta_ref[0]
    a = m * af_ref[...] + (1.0 - m) * as_ref[...]
    ca_ref[...] = a.astype(_CA_DTYPE)
    z1_ref[...] = jnp.dot(a, t1_scr[...], preferred_element_type=jnp.float32)


def _p2_body(z1_ref, w2_ref, ca_ref, z2_ref, h_scr):
    @pl.when(pl.program_id(0) == 0)
    def _():
        h_scr[...] = jnp.tanh(
            jnp.dot(z1_ref[...], w2_ref[...], preferred_element_type=jnp.float32))

    z2_ref[...] = jnp.dot(ca_ref[...].astype(jnp.float32), h_scr[...],
                          preferred_element_type=jnp.float32)


def _p3_body(z2_ref, w3_ref, ca_ref, zf_ref, zs_ref, wl_w_ref, wl_b_ref,
             mlp_w_ref, mlp_b_ref, out_ref, h_scr):
    @pl.when(pl.program_id(0) == 0)
    def _():
        h_scr[...] = jnp.dot(z2_ref[...], w3_ref[...],
                             preferred_element_type=jnp.float32)

    com = jnp.dot(ca_ref[...].astype(jnp.float32), h_scr[...],
                  preferred_element_type=jnp.float32)
    zf = zf_ref[...]
    zs = zs_ref[...]
    wl = wl_w_ref[...]
    wlb = wl_b_ref[...]
    a0 = jnp.dot(zf, wl, preferred_element_type=jnp.float32) + wlb
    a1 = jnp.dot(com, wl, preferred_element_type=jnp.float32) + wlb
    a2 = jnp.dot(zs, wl, preferred_element_type=jnp.float32) + wlb
    inv = 1.0 / jnp.maximum(jnp.sqrt(a0 * a0 + a1 * a1 + a2 * a2), 1e-12)
    nz = zf.shape[1]
    mw = mlp_w_ref[...]
    out = jnp.dot(a0 * inv * zf, mw[0:nz], preferred_element_type=jnp.float32)
    out += jnp.dot(a1 * inv * com, mw[nz:2 * nz], preferred_element_type=jnp.float32)
    out += jnp.dot(a2 * inv * zs, mw[2 * nz:3 * nz], preferred_element_type=jnp.float32)
    out_ref[...] = out + mlp_b_ref[...]


def kernel(z_feature, z_spatial, adj_feature, adj_spatial, x,
           meta, W1, W2, W3, wl_W, wl_b, mlp_W, mlp_b):
    n, _ = adj_feature.shape
    n_in = x.shape[1]
    e1 = W1.shape[1]
    nz = W3.shape[1]
    wl_b2 = wl_b.reshape(1, -1)
    mlp_b2 = mlp_b.reshape(1, -1)

    g1 = pl.cdiv(n, _BM1)
    z1, con_adj = pl.pallas_call(
        _p1_body,
        grid=(g1,),
        in_specs=[
            pl.BlockSpec(memory_space=pltpu.SMEM),
            pl.BlockSpec((n, n_in), lambda i: (0, 0)),
            pl.BlockSpec((n_in, e1), lambda i: (0, 0)),
            pl.BlockSpec((_BM1, n), lambda i: (i, 0)),
            pl.BlockSpec((_BM1, n), lambda i: (i, 0)),
        ],
        out_specs=[
            pl.BlockSpec((_BM1, e1), lambda i: (i, 0)),
            pl.BlockSpec((_BM1, n), lambda i: (i, 0)),
        ],
        out_shape=[
            jax.ShapeDtypeStruct((n, e1), jnp.float32),
            jax.ShapeDtypeStruct((n, n), _CA_DTYPE),
        ],
        scratch_shapes=[pltpu.VMEM((n, e1), jnp.float32)],
    )(meta, x, W1, adj_feature, adj_spatial)

    e2 = W2.shape[1]
    g2 = pl.cdiv(n, _BM2)
    z2 = pl.pallas_call(
        _p2_body,
        grid=(g2,),
        in_specs=[
            pl.BlockSpec((n, e1), lambda i: (0, 0)),
            pl.BlockSpec((e1, e2), lambda i: (0, 0)),
            pl.BlockSpec((_BM2, n), lambda i: (i, 0)),
        ],
        out_specs=pl.BlockSpec((_BM2, e2), lambda i: (i, 0)),
        out_shape=jax.ShapeDtypeStruct((n, e2), jnp.float32),
        scratch_shapes=[pltpu.VMEM((n, e2), jnp.float32)],
    )(z1, W2, con_adj)

    emb = pl.pallas_call(
        _p3_body,
        grid=(g2,),
        in_specs=[
            pl.BlockSpec((n, e2), lambda i: (0, 0)),
            pl.BlockSpec((e2, nz), lambda i: (0, 0)),
            pl.BlockSpec((_BM2, n), lambda i: (i, 0)),
            pl.BlockSpec((_BM2, nz), lambda i: (i, 0)),
            pl.BlockSpec((_BM2, nz), lambda i: (i, 0)),
            pl.BlockSpec((nz, nz), lambda i: (0, 0)),
            pl.BlockSpec((1, nz), lambda i: (0, 0)),
            pl.BlockSpec((3 * nz, nz), lambda i: (0, 0)),
            pl.BlockSpec((1, nz), lambda i: (0, 0)),
        ],
        out_specs=pl.BlockSpec((_BM2, nz), lambda i: (i, 0)),
        out_shape=jax.ShapeDtypeStruct((n, nz), jnp.float32),
        scratch_shapes=[pltpu.VMEM((n, nz), jnp.float32)],
    )(z2, W3, con_adj, z_feature, z_spatial, wl_W, wl_b2, mlp_W, mlp_b2)

    return emb


# speedup vs baseline: 1.2831x; 1.2831x over previous
import jax
import jax.numpy as jnp
from jax.experimental import pallas as pl
from jax.experimental.pallas import tpu as pltpu

_BM1 = 128
_BM2 = 256
_CA_DTYPE = jnp.bfloat16


def _p1_body(meta_ref, x_ref, w1_ref, af_ref, as_ref, z1_ref, ca_ref, t1_scr):
    @pl.when(pl.program_id(0) == 0)
    def _():
        t1_scr[...] = jnp.tanh(
            jnp.dot(x_ref[...], w1_ref[...],
                    preferred_element_type=jnp.float32)).astype(jnp.bfloat16)

    m = meta_ref[0]
    a = (m * af_ref[...] + (1.0 - m) * as_ref[...]).astype(_CA_DTYPE)
    ca_ref[...] = a
    z1_ref[...] = jnp.dot(a, t1_scr[...], preferred_element_type=jnp.float32)


def _p2_body(z1_ref, w2_ref, ca_ref, z2_ref, h_scr):
    @pl.when(pl.program_id(0) == 0)
    def _():
        h_scr[...] = jnp.tanh(
            jnp.dot(z1_ref[...], w2_ref[...],
                    preferred_element_type=jnp.float32)).astype(jnp.bfloat16)

    z2_ref[...] = jnp.dot(ca_ref[...], h_scr[...],
                          preferred_element_type=jnp.float32)


def _p3_body(z2_ref, w3_ref, ca_ref, zf_ref, zs_ref, wl_w_ref, wl_b_ref,
             mlp_w_ref, mlp_b_ref, out_ref, h_scr):
    @pl.when(pl.program_id(0) == 0)
    def _():
        h_scr[...] = jnp.dot(z2_ref[...], w3_ref[...],
                             preferred_element_type=jnp.float32).astype(jnp.bfloat16)

    com = jnp.dot(ca_ref[...], h_scr[...],
                  preferred_element_type=jnp.float32)
    zf = zf_ref[...]
    zs = zs_ref[...]
    wl = wl_w_ref[...]
    wlb = wl_b_ref[...]
    a0 = jnp.dot(zf, wl, preferred_element_type=jnp.float32) + wlb
    a1 = jnp.dot(com, wl, preferred_element_type=jnp.float32) + wlb
    a2 = jnp.dot(zs, wl, preferred_element_type=jnp.float32) + wlb
    inv = 1.0 / jnp.maximum(jnp.sqrt(a0 * a0 + a1 * a1 + a2 * a2), 1e-12)
    nz = zf.shape[1]
    mw = mlp_w_ref[...]
    out = jnp.dot(a0 * inv * zf, mw[0:nz], preferred_element_type=jnp.float32)
    out += jnp.dot(a1 * inv * com, mw[nz:2 * nz], preferred_element_type=jnp.float32)
    out += jnp.dot(a2 * inv * zs, mw[2 * nz:3 * nz], preferred_element_type=jnp.float32)
    out_ref[...] = out + mlp_b_ref[...]


def kernel(z_feature, z_spatial, adj_feature, adj_spatial, x,
           meta, W1, W2, W3, wl_W, wl_b, mlp_W, mlp_b):
    n, _ = adj_feature.shape
    n_in = x.shape[1]
    e1 = W1.shape[1]
    nz = W3.shape[1]
    wl_b2 = wl_b.reshape(1, -1)
    mlp_b2 = mlp_b.reshape(1, -1)

    g1 = pl.cdiv(n, _BM1)
    z1, con_adj = pl.pallas_call(
        _p1_body,
        grid=(g1,),
        in_specs=[
            pl.BlockSpec(memory_space=pltpu.SMEM),
            pl.BlockSpec((n, n_in), lambda i: (0, 0)),
            pl.BlockSpec((n_in, e1), lambda i: (0, 0)),
            pl.BlockSpec((_BM1, n), lambda i: (i, 0)),
            pl.BlockSpec((_BM1, n), lambda i: (i, 0)),
        ],
        out_specs=[
            pl.BlockSpec((_BM1, e1), lambda i: (i, 0)),
            pl.BlockSpec((_BM1, n), lambda i: (i, 0)),
        ],
        out_shape=[
            jax.ShapeDtypeStruct((n, e1), jnp.float32),
            jax.ShapeDtypeStruct((n, n), _CA_DTYPE),
        ],
        scratch_shapes=[pltpu.VMEM((n, e1), jnp.bfloat16)],
    )(meta, x, W1, adj_feature, adj_spatial)

    e2 = W2.shape[1]
    g2 = pl.cdiv(n, _BM2)
    z2 = pl.pallas_call(
        _p2_body,
        grid=(g2,),
        in_specs=[
            pl.BlockSpec((n, e1), lambda i: (0, 0)),
            pl.BlockSpec((e1, e2), lambda i: (0, 0)),
            pl.BlockSpec((_BM2, n), lambda i: (i, 0)),
        ],
        out_specs=pl.BlockSpec((_BM2, e2), lambda i: (i, 0)),
        out_shape=jax.ShapeDtypeStruct((n, e2), jnp.float32),
        scratch_shapes=[pltpu.VMEM((n, e2), jnp.bfloat16)],
    )(z1, W2, con_adj)

    emb = pl.pallas_call(
        _p3_body,
        grid=(g2,),
        in_specs=[
            pl.BlockSpec((n, e2), lambda i: (0, 0)),
            pl.BlockSpec((e2, nz), lambda i: (0, 0)),
            pl.BlockSpec((_BM2, n), lambda i: (i, 0)),
            pl.BlockSpec((_BM2, nz), lambda i: (i, 0)),
            pl.BlockSpec((_BM2, nz), lambda i: (i, 0)),
            pl.BlockSpec((nz, nz), lambda i: (0, 0)),
            pl.BlockSpec((1, nz), lambda i: (0, 0)),
            pl.BlockSpec((3 * nz, nz), lambda i: (0, 0)),
            pl.BlockSpec((1, nz), lambda i: (0, 0)),
        ],
        out_specs=pl.BlockSpec((_BM2, nz), lambda i: (i, 0)),
        out_shape=jax.ShapeDtypeStruct((n, nz), jnp.float32),
        scratch_shapes=[pltpu.VMEM((n, nz), jnp.bfloat16)],
    )(z2, W3, con_adj, z_feature, z_spatial, wl_W, wl_b2, mlp_W, mlp_b2)

    return emb


# speedup vs baseline: 1.3330x; 1.0389x over previous
import jax
import jax.numpy as jnp
from jax.experimental import pallas as pl
from jax.experimental.pallas import tpu as pltpu

_BM1 = 192
_BM2 = 512
_CPARAMS = pltpu.CompilerParams(
    dimension_semantics=("arbitrary",),
    vmem_limit_bytes=112 * 1024 * 1024,
)
_CA_DTYPE = jnp.bfloat16


def _p1_body(meta_ref, x_ref, w1_ref, af_ref, as_ref, z1_ref, ca_ref, t1_scr):
    @pl.when(pl.program_id(0) == 0)
    def _():
        t1_scr[...] = jnp.tanh(
            jnp.dot(x_ref[...], w1_ref[...],
                    preferred_element_type=jnp.float32)).astype(jnp.bfloat16)

    m = meta_ref[0]
    a = (m * af_ref[...] + (1.0 - m) * as_ref[...]).astype(_CA_DTYPE)
    ca_ref[...] = a
    z1_ref[...] = jnp.dot(a, t1_scr[...], preferred_element_type=jnp.float32)


def _p2_body(z1_ref, w2_ref, ca_ref, z2_ref, h_scr):
    @pl.when(pl.program_id(0) == 0)
    def _():
        h_scr[...] = jnp.tanh(
            jnp.dot(z1_ref[...], w2_ref[...],
                    preferred_element_type=jnp.float32)).astype(jnp.bfloat16)

    z2_ref[...] = jnp.dot(ca_ref[...], h_scr[...],
                          preferred_element_type=jnp.float32)


def _p3_body(z2_ref, w3_ref, ca_ref, zf_ref, zs_ref, wl_w_ref, wl_b_ref,
             mlp_w_ref, mlp_b_ref, out_ref, h_scr):
    @pl.when(pl.program_id(0) == 0)
    def _():
        h_scr[...] = jnp.dot(z2_ref[...], w3_ref[...],
                             preferred_element_type=jnp.float32).astype(jnp.bfloat16)

    com = jnp.dot(ca_ref[...], h_scr[...],
                  preferred_element_type=jnp.float32)
    zf = zf_ref[...]
    zs = zs_ref[...]
    wl = wl_w_ref[...]
    wlb = wl_b_ref[...]
    a0 = jnp.dot(zf, wl, preferred_element_type=jnp.float32) + wlb
    a1 = jnp.dot(com, wl, preferred_element_type=jnp.float32) + wlb
    a2 = jnp.dot(zs, wl, preferred_element_type=jnp.float32) + wlb
    inv = 1.0 / jnp.maximum(jnp.sqrt(a0 * a0 + a1 * a1 + a2 * a2), 1e-12)
    nz = zf.shape[1]
    mw = mlp_w_ref[...]
    out = jnp.dot(a0 * inv * zf, mw[0:nz], preferred_element_type=jnp.float32)
    out += jnp.dot(a1 * inv * com, mw[nz:2 * nz], preferred_element_type=jnp.float32)
    out += jnp.dot(a2 * inv * zs, mw[2 * nz:3 * nz], preferred_element_type=jnp.float32)
    out_ref[...] = out + mlp_b_ref[...]


def kernel(z_feature, z_spatial, adj_feature, adj_spatial, x,
           meta, W1, W2, W3, wl_W, wl_b, mlp_W, mlp_b):
    n, _ = adj_feature.shape
    n_in = x.shape[1]
    e1 = W1.shape[1]
    nz = W3.shape[1]
    wl_b2 = wl_b.reshape(1, -1)
    mlp_b2 = mlp_b.reshape(1, -1)

    g1 = pl.cdiv(n, _BM1)
    z1, con_adj = pl.pallas_call(
        _p1_body,
        grid=(g1,),
        in_specs=[
            pl.BlockSpec(memory_space=pltpu.SMEM),
            pl.BlockSpec((n, n_in), lambda i: (0, 0)),
            pl.BlockSpec((n_in, e1), lambda i: (0, 0)),
            pl.BlockSpec((_BM1, n), lambda i: (i, 0)),
            pl.BlockSpec((_BM1, n), lambda i: (i, 0)),
        ],
        out_specs=[
            pl.BlockSpec((_BM1, e1), lambda i: (i, 0)),
            pl.BlockSpec((_BM1, n), lambda i: (i, 0)),
        ],
        out_shape=[
            jax.ShapeDtypeStruct((n, e1), jnp.float32),
            jax.ShapeDtypeStruct((n, n), _CA_DTYPE),
        ],
        scratch_shapes=[pltpu.VMEM((n, e1), jnp.bfloat16)],
        compiler_params=_CPARAMS,
    )(meta, x, W1, adj_feature, adj_spatial)

    e2 = W2.shape[1]
    g2 = pl.cdiv(n, _BM2)
    z2 = pl.pallas_call(
        _p2_body,
        grid=(g2,),
        in_specs=[
            pl.BlockSpec((n, e1), lambda i: (0, 0)),
            pl.BlockSpec((e1, e2), lambda i: (0, 0)),
            pl.BlockSpec((_BM2, n), lambda i: (i, 0)),
        ],
        out_specs=pl.BlockSpec((_BM2, e2), lambda i: (i, 0)),
        out_shape=jax.ShapeDtypeStruct((n, e2), jnp.float32),
        scratch_shapes=[pltpu.VMEM((n, e2), jnp.bfloat16)],
        compiler_params=_CPARAMS,
    )(z1, W2, con_adj)

    emb = pl.pallas_call(
        _p3_body,
        grid=(g2,),
        in_specs=[
            pl.BlockSpec((n, e2), lambda i: (0, 0)),
            pl.BlockSpec((e2, nz), lambda i: (0, 0)),
            pl.BlockSpec((_BM2, n), lambda i: (i, 0)),
            pl.BlockSpec((_BM2, nz), lambda i: (i, 0)),
            pl.BlockSpec((_BM2, nz), lambda i: (i, 0)),
            pl.BlockSpec((nz, nz), lambda i: (0, 0)),
            pl.BlockSpec((1, nz), lambda i: (0, 0)),
            pl.BlockSpec((3 * nz, nz), lambda i: (0, 0)),
            pl.BlockSpec((1, nz), lambda i: (0, 0)),
        ],
        out_specs=pl.BlockSpec((_BM2, nz), lambda i: (i, 0)),
        out_shape=jax.ShapeDtypeStruct((n, nz), jnp.float32),
        scratch_shapes=[pltpu.VMEM((n, nz), jnp.bfloat16)],
        compiler_params=_CPARAMS,
    )(z2, W3, con_adj, z_feature, z_spatial, wl_W, wl_b2, mlp_W, mlp_b2)

    return emb
